# Initial kernel scaffold; baseline (speedup 1.0000x reference)
import jax
import jax.numpy as jnp
from jax.experimental import pallas as pl


def kernel(input_ids, base_table, sense_table, proj_W):
    raise NotImplementedError("write your pallas kernel here")


# speedup vs baseline: 6.3450x; 6.3450x over previous
import functools

import jax
import jax.numpy as jnp
from jax import lax
from jax.experimental import pallas as pl
from jax.experimental.pallas import tpu as pltpu
from jax.experimental.pallas import tpu_sc as plsc

V = 100000
D = 64
ROWS_BLK = 2000
N_BLK = 2 * V // ROWS_BLK

_B, _L = 4096, 200
BL = _B * _L
NW = 32
PER_W = BL // NW
CHUNK = 128
GROUP = 8
TOK_GRP = CHUNK * GROUP
N_GRP = PER_W // TOK_GRP
IDX_ROWS = PER_W // CHUNK


def _combined_body(base_ref, sense_ref, w_ref, out_ref):
    i = pl.program_id(0)

    @pl.when(i < N_BLK // 2)
    def _copy():
        out_ref[...] = base_ref[...]

    @pl.when(i >= N_BLK // 2)
    def _proj():
        out_ref[...] = jnp.dot(sense_ref[...], w_ref[...],
                               preferred_element_type=jnp.float32)


def _build_combined(base_table, sense_table, proj_W):
    h = N_BLK // 2
    return pl.pallas_call(
        _combined_body,
        grid=(N_BLK,),
        in_specs=[
            pl.BlockSpec((ROWS_BLK, D), lambda i: (jnp.minimum(i, h - 1), 0)),
            pl.BlockSpec((ROWS_BLK, D), lambda i: (jnp.maximum(i - h, 0), 0)),
            pl.BlockSpec((D, D), lambda i: (0, 0)),
        ],
        out_specs=pl.BlockSpec((ROWS_BLK, D), lambda i: (i, 0)),
        out_shape=jax.ShapeDtypeStruct((2 * V, D), jnp.float32),
    )(base_table, sense_table, proj_W)


def _make_gather():
    mesh = plsc.VectorSubcoreMesh(core_axis_name="c", subcore_axis_name="s")

    @functools.partial(
        pl.kernel,
        mesh=mesh,
        compiler_params=pltpu.CompilerParams(use_tc_tiling_on_sc=False),
        out_type=jax.ShapeDtypeStruct((BL, D), jnp.float32),
        scratch_types=[
            pltpu.VMEM((IDX_ROWS, CHUNK), jnp.int32),
            pltpu.VMEM((TOK_GRP, D), jnp.float32),
            pltpu.SemaphoreType.DMA,
        ],
    )
    def gather_k(ids_hbm, table_hbm, out_hbm, idx_v, rows_v, sem):
        wid = lax.axis_index("s") * 2 + lax.axis_index("c")
        pltpu.sync_copy(ids_hbm.at[pl.ds(wid * IDX_ROWS, IDX_ROWS)], idx_v)

        def body(g, carry):
            base = wid * PER_W + g * TOK_GRP
            cps = [
                pltpu.async_copy(
                    table_hbm.at[idx_v.at[g * GROUP + b]],
                    rows_v.at[pl.ds(b * CHUNK, CHUNK)],
                    sem,
                )
                for b in range(GROUP)
            ]
            for c in cps:
                c.wait()
            pltpu.sync_copy(rows_v, out_hbm.at[pl.ds(base, TOK_GRP)])
            return carry

        lax.fori_loop(0, N_GRP, body, 0)

    return gather_k


_gather_cache = []


def kernel(input_ids, base_table, sense_table, proj_W):
    if not _gather_cache:
        _gather_cache.append(_make_gather())
    combined = _build_combined(base_table, sense_table, proj_W)
    ids = input_ids.reshape(BL // CHUNK, CHUNK).astype(jnp.int32)
    out = _gather_cache[0](ids, combined)
    return out.reshape(_B, _L, D)


# speedup vs baseline: 6.4362x; 1.0144x over previous
import functools

import jax
import jax.numpy as jnp
from jax import lax
from jax.experimental import pallas as pl
from jax.experimental.pallas import tpu as pltpu
from jax.experimental.pallas import tpu_sc as plsc

V = 100000
D = 64
ROWS_BLK = 2000
N_BLK = 2 * V // ROWS_BLK

_B, _L = 4096, 200
BL = _B * _L
NW = 32
PER_W = BL // NW
CHUNK = 128
GROUP = 4
TOK_GRP = CHUNK * GROUP
N_GRP = PER_W // TOK_GRP
IDX_ROWS = PER_W // CHUNK


def _combined_body(base_ref, sense_ref, w_ref, out_ref):
    i = pl.program_id(0)

    @pl.when(i < N_BLK // 2)
    def _copy():
        out_ref[...] = base_ref[...]

    @pl.when(i >= N_BLK // 2)
    def _proj():
        out_ref[...] = jnp.dot(sense_ref[...], w_ref[...],
                               preferred_element_type=jnp.float32)


def _build_combined(base_table, sense_table, proj_W):
    h = N_BLK // 2
    return pl.pallas_call(
        _combined_body,
        grid=(N_BLK,),
        in_specs=[
            pl.BlockSpec((ROWS_BLK, D), lambda i: (jnp.minimum(i, h - 1), 0)),
            pl.BlockSpec((ROWS_BLK, D), lambda i: (jnp.maximum(i - h, 0), 0)),
            pl.BlockSpec((D, D), lambda i: (0, 0)),
        ],
        out_specs=pl.BlockSpec((ROWS_BLK, D), lambda i: (i, 0)),
        out_shape=jax.ShapeDtypeStruct((2 * V, D), jnp.float32),
    )(base_table, sense_table, proj_W)


def _make_gather():
    mesh = plsc.VectorSubcoreMesh(core_axis_name="c", subcore_axis_name="s")

    @functools.partial(
        pl.kernel,
        mesh=mesh,
        compiler_params=pltpu.CompilerParams(use_tc_tiling_on_sc=False),
        out_type=jax.ShapeDtypeStruct((BL, D), jnp.float32),
        scratch_types=[
            pltpu.VMEM((IDX_ROWS, CHUNK), jnp.int32),
            pltpu.VMEM((2, TOK_GRP, D), jnp.float32),
            pltpu.SemaphoreType.DMA,
            pltpu.SemaphoreType.DMA,
        ],
    )
    def gather_k(ids_hbm, table_hbm, out_hbm, idx_v, rows_v, gsem, wsem):
        wid = lax.axis_index("s") * 2 + lax.axis_index("c")
        pltpu.sync_copy(ids_hbm.at[pl.ds(wid * IDX_ROWS, IDX_ROWS)], idx_v)

        def issue_gathers(g, slot):
            for b in range(GROUP):
                pltpu.async_copy(
                    table_hbm.at[idx_v.at[g * GROUP + b]],
                    rows_v.at[slot].at[pl.ds(b * CHUNK, CHUNK)],
                    gsem,
                )

        def wait_gathers(slot):
            pltpu.make_async_copy(
                table_hbm.at[pl.ds(0, TOK_GRP)], rows_v.at[slot], gsem
            ).wait()

        def issue_wb(g, slot):
            base = wid * PER_W + g * TOK_GRP
            pltpu.async_copy(rows_v.at[slot], out_hbm.at[pl.ds(base, TOK_GRP)], wsem)

        def wait_wb(slot):
            pltpu.make_async_copy(
                rows_v.at[slot], out_hbm.at[pl.ds(0, TOK_GRP)], wsem
            ).wait()

        issue_gathers(0, 0)

        def body(g, carry):
            slot = lax.rem(g, 2)
            prev = lax.rem(g + 1, 2)

            @pl.when(g < N_GRP)
            def _():
                @pl.when(g >= 2)
                def _():
                    wait_wb(slot)
                issue_gathers(g, slot)

            wait_gathers(prev)
            issue_wb(g - 1, prev)
            return carry

        lax.fori_loop(1, N_GRP + 1, body, 0)
        wait_wb(lax.rem(N_GRP - 1, 2))
        wait_wb(lax.rem(N_GRP, 2))

    return gather_k


_gather_cache = []


def kernel(input_ids, base_table, sense_table, proj_W):
    if not _gather_cache:
        _gather_cache.append(_make_gather())
    combined = _build_combined(base_table, sense_table, proj_W)
    ids = input_ids.reshape(BL // CHUNK, CHUNK).astype(jnp.int32)
    out = _gather_cache[0](ids, combined)
    return out.reshape(_B, _L, D)
